# baseline (speedup 1.0000x reference)
import math

import jax
import jax.numpy as jnp
from jax import lax
from jax.experimental import pallas as pl
from jax.experimental.pallas import tpu as pltpu
from jax.experimental.pallas import tpu_sc as plsc

N = 10000
E = 320000
FEAT = 128
IN_DIM = 7
INST = 20
F_CAT = FEAT * 3 + INST
SIGMA = 25.0

NC = 2
NS = 16
LANES = 16
NW = NC * NS

EW = E // NW
GCH = 80
NGCH = EW // GCH

NB = 320
NPAD = NW * NB
SUB = 128
WCH = 2000
PAGE = 512
PITCH = PAGE + 16
CAPB = 10240
QMUL = 6554
QSH = 21

_ROWBLK = 2000
_EBLK = 4000


def _p0_body(t_ref, x_ref, wg_ref, tw_ref, tb_ref, xw_ref, xb_ref,
             te_ref, h0_ref, sc_ref):
    t = t_ref[...]
    proj = t * wg_ref[...] * (2.0 * math.pi)
    gf = jnp.concatenate([jnp.sin(proj), jnp.cos(proj)], axis=1)
    te = jnp.dot(gf, tw_ref[...], preferred_element_type=jnp.float32) + tb_ref[...]
    te_ref[...] = te * jax.nn.sigmoid(te)
    h0_ref[...] = (jnp.dot(x_ref[...], xw_ref[...],
                           preferred_element_type=jnp.float32) + xb_ref[...])
    ln2 = 2.0 * math.log(SIGMA)
    std = jnp.sqrt((jnp.exp(t * ln2) - 1.0) / ln2)
    sc_ref[...] = jnp.broadcast_to(1.0 / (std + 1e-7), (t.shape[0], LANES))


def _p0_call(t, x, wg, tw, tb, xw, xb):
    grid = (N // _ROWBLK,)
    full = lambda shape: pl.BlockSpec(shape, lambda i: (0, 0))
    row = lambda w: pl.BlockSpec((_ROWBLK, w), lambda i: (i, 0))
    return pl.pallas_call(
        _p0_body,
        grid=grid,
        in_specs=[row(1), row(IN_DIM), full((1, FEAT // 2)),
                  full((FEAT, FEAT)), full((1, FEAT)),
                  full((IN_DIM, FEAT)), full((1, FEAT))],
        out_specs=[row(FEAT), row(FEAT), row(LANES)],
        out_shape=[jax.ShapeDtypeStruct((N, FEAT), jnp.float32),
                   jax.ShapeDtypeStruct((N, FEAT), jnp.float32),
                   jax.ShapeDtypeStruct((N, LANES), jnp.float32)],
    )(t, x, wg, tw, tb, xw, xb)


def _ab_body(h_ref, te_ref, pp_ref, il_ref,
             wdh_ref, wdt_ref, wdp_ref, wdi_ref, b1_ref,
             wbh_ref, wbt_ref, wbp_ref, wbi_ref,
             a_ref, b_ref):
    h = h_ref[...]
    te = te_ref[...]
    pp = pp_ref[...]
    il = il_ref[...]
    dot = lambda a, w: jnp.dot(a, w[...], preferred_element_type=jnp.float32)
    a_ref[...] = (dot(h, wdh_ref) + dot(te, wdt_ref) + dot(pp, wdp_ref)
                  + dot(il, wdi_ref) + b1_ref[...])
    b_ref[...] = (dot(h, wbh_ref) + dot(te, wbt_ref) + dot(pp, wbp_ref)
                  + dot(il, wbi_ref))


def _ab_call(h, te, pp, il, wdh, wdt, wdp, wdi, b1, wbh, wbt, wbp, wbi):
    grid = (N // _ROWBLK,)
    full = lambda shape: pl.BlockSpec(shape, lambda i: (0, 0))
    row = lambda w: pl.BlockSpec((_ROWBLK, w), lambda i: (i, 0))
    return pl.pallas_call(
        _ab_body,
        grid=grid,
        in_specs=[row(FEAT), row(FEAT), row(FEAT), row(INST),
                  full((FEAT, FEAT)), full((FEAT, FEAT)), full((FEAT, FEAT)),
                  full((INST, FEAT)), full((1, FEAT)),
                  full((FEAT, FEAT)), full((FEAT, FEAT)), full((FEAT, FEAT)),
                  full((INST, FEAT))],
        out_specs=[row(FEAT), row(FEAT)],
        out_shape=[jax.ShapeDtypeStruct((N, FEAT), jnp.float32),
                   jax.ShapeDtypeStruct((N, FEAT), jnp.float32)],
    )(h, te, pp, il, wdh, wdt, wdp, wdi, b1, wbh, wbt, wbp, wbi)


def _mm_body(p_ref, w2_ref, b2_ref, m_ref):
    m_ref[...] = (jnp.dot(p_ref[...], w2_ref[...],
                          preferred_element_type=jnp.float32) + b2_ref[...])


def _mm_call(p, w2, b2, doutp):
    grid = (E // _EBLK,)
    return pl.pallas_call(
        _mm_body,
        grid=grid,
        in_specs=[pl.BlockSpec((_EBLK, FEAT), lambda i: (i, 0)),
                  pl.BlockSpec((FEAT, doutp), lambda i: (0, 0)),
                  pl.BlockSpec((1, doutp), lambda i: (0, 0))],
        out_specs=pl.BlockSpec((_EBLK, doutp), lambda i: (i, 0)),
        out_shape=jax.ShapeDtypeStruct((E, doutp), jnp.float32),
    )(p, w2, b2)


def _gather_body(a_hbm, b_hbm, dst_hbm, src_hbm, p_hbm,
                 idxd_v, idxs_v, rowsa_v, rowsb_v, sem):
    wid = lax.axis_index("s") * NC + lax.axis_index("c")
    base = wid * EW

    def chunk(ch, carry):
        off = base + ch * GCH
        pltpu.sync_copy(dst_hbm.at[pl.ds(off, GCH)], idxd_v)
        pltpu.sync_copy(src_hbm.at[pl.ds(off, GCH)], idxs_v)
        da = pltpu.async_copy(a_hbm.at[idxd_v], rowsa_v, sem)
        db = pltpu.async_copy(b_hbm.at[idxs_v], rowsb_v, sem)
        da.wait()
        db.wait()

        def row(r, c2):
            for c in range(FEAT // LANES):
                s = pl.ds(c * LANES, LANES)
                rowsa_v[r, s] = jnp.maximum(rowsa_v[r, s] + rowsb_v[r, s], 0.0)
            return c2
        lax.fori_loop(0, GCH, row, 0)
        pltpu.sync_copy(rowsa_v, p_hbm.at[pl.ds(off, GCH)])
        return carry
    lax.fori_loop(0, NGCH, chunk, 0)


def _make_gather():
    mesh = plsc.VectorSubcoreMesh(core_axis_name="c", subcore_axis_name="s",
                                  num_cores=NC, num_subcores=NS)
    return pl.kernel(
        _gather_body,
        out_type=jax.ShapeDtypeStruct((E, FEAT), jnp.float32),
        mesh=mesh,
        scratch_types=[
            pltpu.VMEM((GCH,), jnp.int32),
            pltpu.VMEM((GCH,), jnp.int32),
            pltpu.VMEM((GCH, FEAT), jnp.float32),
            pltpu.VMEM((GCH, FEAT), jnp.float32),
            pltpu.SemaphoreType.DMA,
        ],
    )


def _bucket_body(dst_hbm, ex_hbm, cnt_hbm, dbuf, staging, cntbuf):
    wid = lax.axis_index("s") * NC + lax.axis_index("c")
    ebase = wid * EW
    iota = lax.iota(jnp.int32, LANES)
    zeros16 = jnp.zeros((LANES,), jnp.int32)
    for c in range(3):
        cntbuf[pl.ds(c * LANES, LANES)] = zeros16

    def chunk(ch, carry):
        pltpu.sync_copy(dst_hbm.at[pl.ds(ebase + ch * WCH, WCH)], dbuf)

        def group(g, c2):
            d16 = dbuf[pl.ds(g * LANES, LANES)]
            q16 = (d16 * QMUL) >> QSH
            dloc16 = d16 - q16 * NB
            e0 = ebase + ch * WCH + g * LANES
            pk16 = dloc16 + (e0 + iota) * 512
            for l in range(LANES):
                b = q16[l]
                pk = zeros16 + pk16[l]
                cb = cntbuf[pl.ds(b, LANES)][0]
                pos = cb & (PAGE - 1)
                staging[pl.ds(b * PITCH + pos, LANES)] = pk
                bb = (b >> 4) << 4
                cv = cntbuf[pl.ds(bb, LANES)]
                cntbuf[pl.ds(bb, LANES)] = cv + jnp.where(
                    iota == b - bb, jnp.ones((LANES,), jnp.int32), zeros16)

                @pl.when(pos == PAGE - 1)
                def _flush():
                    pltpu.sync_copy(
                        staging.at[pl.ds(b * PITCH, PAGE)],
                        ex_hbm.at[pl.ds(pl.multiple_of(
                            (wid * NW + b) * CAPB + cb - (PAGE - 1), 8),
                            PAGE)])
            return c2
        lax.fori_loop(0, WCH // LANES, group, 0)
        return carry
    lax.fori_loop(0, EW // WCH, chunk, 0)

    for b in range(NW):
        cb = cntbuf[pl.ds(b, LANES)][0]
        base = (cb >> 9) << 9
        pltpu.sync_copy(
            staging.at[pl.ds(b * PITCH, PAGE)],
            ex_hbm.at[pl.ds(pl.multiple_of(
                (wid * NW + b) * CAPB + base, 8), PAGE)])
    pltpu.sync_copy(cntbuf.at[pl.ds(0, NW)],
                    cnt_hbm.at[pl.ds(pl.multiple_of(wid * NW, 8), NW)])


def _make_bucket():
    mesh = plsc.VectorSubcoreMesh(core_axis_name="c", subcore_axis_name="s",
                                  num_cores=NC, num_subcores=NS)
    return pl.kernel(
        _bucket_body,
        out_type=(jax.ShapeDtypeStruct((NW * NW * CAPB,), jnp.int32),
                  jax.ShapeDtypeStruct((NW * NW,), jnp.int32)),
        mesh=mesh,
        scratch_types=[
            pltpu.VMEM((WCH,), jnp.int32),
            pltpu.VMEM((NW * PITCH,), jnp.int32),
            pltpu.VMEM((3 * LANES,), jnp.int32),
        ],
    )


def _consumer_body(doutp, do_relu, use_scale, *refs):
    if use_scale:
        (m_hbm, ex_hbm, cnt_hbm, sc_hbm, o_hbm,
         cntv, pkbuf, idxbuf, dlb, rows_v, acc_v, acc_w, scv, sem) = refs
    else:
        (m_hbm, ex_hbm, cnt_hbm, o_hbm,
         cntv, pkbuf, idxbuf, dlb, rows_v, acc_v, acc_w, sem) = refs
    wid = lax.axis_index("s") * NC + lax.axis_index("c")
    lo = wid * NB
    cg = doutp // LANES
    iota = lax.iota(jnp.int32, LANES)
    neg = jnp.full((LANES,), -jnp.inf, jnp.float32)

    def initr(r, c2):
        acc_v[pl.ds(r * LANES, LANES)] = neg
        acc_w[pl.ds(r * LANES, LANES)] = neg
        return c2
    lax.fori_loop(0, (NB + 1) * cg, initr, 0)

    pltpu.sync_copy(cnt_hbm, cntv.at[pl.ds(0, NW * NW)])

    def writer(t, c2):
        ct = cntv[pl.ds(t * NW + wid, LANES)][0]
        npages = (ct + PAGE - 1) >> 9

        def page(pg, c3):
            pltpu.sync_copy(
                ex_hbm.at[pl.ds(pl.multiple_of(
                    (t * NW + wid) * CAPB + pg * PAGE, 8), PAGE)],
                pkbuf)
            nval = jnp.minimum(ct - pg * PAGE, PAGE)
            nsub = (nval + SUB - 1) >> 7

            def sub(sb, c4):
                base = sb * SUB
                valid = nval - base
                for gg in range(SUB // LANES):
                    pk = pkbuf[pl.ds(base + gg * LANES, LANES)]
                    okm = (iota + gg * LANES) < valid
                    idxbuf[pl.ds(gg * LANES, LANES)] = jnp.where(
                        okm, pk >> 9, 0)
                    dlb[pl.ds(gg * LANES, LANES)] = jnp.where(
                        okm, pk & 511, NB)
                pltpu.async_copy(m_hbm.at[idxbuf], rows_v, sem).wait()

                def k8(kk, c5):
                    dlv = dlb[pl.ds(kk * LANES, LANES)]
                    for l in range(LANES):
                        dl = dlv[l]
                        ro = kk * LANES + l
                        ac = acc_v if (l & 1) == 0 else acc_w
                        for c in range(cg):
                            off = dl * doutp + c * LANES
                            cs = pl.ds(c * LANES, LANES)
                            ac[pl.ds(off, LANES)] = jnp.maximum(
                                ac[pl.ds(off, LANES)], rows_v[ro, cs])
                    return c5
                lax.fori_loop(0, SUB // LANES, k8, 0)
                return c4
            lax.fori_loop(0, nsub, sub, 0)
            return c3
        lax.fori_loop(0, npages, page, 0)
        return c2
    lax.fori_loop(0, NW, writer, 0)

    if use_scale:
        pltpu.sync_copy(
            sc_hbm.at[pl.ds(pl.multiple_of(lo * LANES, 8), NB * LANES)], scv)

    def epi(r, c2):
        for c in range(cg):
            off = r * doutp + c * LANES
            v = jnp.maximum(acc_v[pl.ds(off, LANES)],
                            acc_w[pl.ds(off, LANES)])
            ok = jnp.abs(v) < jnp.inf
            v = jnp.where(ok, v, 0.0)
            if do_relu:
                v = jnp.maximum(v, 0.0)
            if use_scale:
                v = v * scv[pl.ds(r * LANES, LANES)]
            acc_v[pl.ds(off, LANES)] = v
        return c2
    lax.fori_loop(0, NB, epi, 0)
    pltpu.sync_copy(acc_v.at[pl.ds(0, NB * doutp)],
                    o_hbm.at[pl.ds(pl.multiple_of(lo * doutp, 8),
                                   NB * doutp)])


def _make_consumer(doutp, do_relu, use_scale):
    mesh = plsc.VectorSubcoreMesh(core_axis_name="c", subcore_axis_name="s",
                                  num_cores=NC, num_subcores=NS)
    scratch = [
        pltpu.VMEM((NW * NW + LANES,), jnp.int32),
        pltpu.VMEM((PAGE,), jnp.int32),
        pltpu.VMEM((SUB,), jnp.int32),
        pltpu.VMEM((SUB,), jnp.int32),
        pltpu.VMEM((SUB, doutp), jnp.float32),
        pltpu.VMEM(((NB + 1) * doutp,), jnp.float32),
        pltpu.VMEM(((NB + 1) * doutp,), jnp.float32),
    ]
    if use_scale:
        scratch.append(pltpu.VMEM((NB * LANES,), jnp.float32))
    scratch.append(pltpu.SemaphoreType.DMA)

    def body(*refs):
        _consumer_body(doutp, do_relu, use_scale, *refs)

    return pl.kernel(
        body,
        out_type=jax.ShapeDtypeStruct((NPAD * doutp,), jnp.float32),
        mesh=mesh,
        scratch_types=scratch,
    )


def kernel(x, t, proc_part_pcs, instance_label, edge_index, W_gfp, tW, tb,
           xW, xb, m1W1, m1b1, m1W2, m1b2, m2W1, m2b1, m2W2, m2b2,
           m3W1, m3b1, m3W2, m3b2):
    src = edge_index[0]
    dst = edge_index[1]
    wg = W_gfp.reshape(1, FEAT // 2)
    te, h0, scale16 = _p0_call(t, x, wg, tW, tb.reshape(1, FEAT),
                               xW, xb.reshape(1, FEAT))
    scale_pad = jnp.pad(scale16, ((0, NPAD - N), (0, 0)))

    gather = _make_gather()
    bucket = _make_bucket()
    cons12 = _make_consumer(FEAT, True, False)
    cons3 = _make_consumer(FEAT, False, True)

    ex, cnt = bucket(dst)
    scale_flat = scale_pad.reshape(-1)

    h = h0
    layers = [(m1W1, m1b1, m1W2, m1b2), (m2W1, m2b1, m2W2, m2b2),
              (m3W1, m3b1, m3W2, m3b2)]
    out = None
    for li, (W1, b1, W2, b2) in enumerate(layers):
        Wa, Wb = W1[:F_CAT], W1[F_CAT:]
        Wd = Wa - Wb
        wdh, wdt, wdp, wdi = (Wd[:FEAT], Wd[FEAT:2 * FEAT],
                              Wd[2 * FEAT:3 * FEAT], Wd[3 * FEAT:])
        wbh, wbt, wbp, wbi = (Wb[:FEAT], Wb[FEAT:2 * FEAT],
                              Wb[2 * FEAT:3 * FEAT], Wb[3 * FEAT:])
        A, B = _ab_call(h, te, proc_part_pcs, instance_label,
                        wdh, wdt, wdp, wdi, b1.reshape(1, FEAT),
                        wbh, wbt, wbp, wbi)
        P = gather(A, B, dst, src)
        if li < 2:
            M = _mm_call(P, W2, b2.reshape(1, FEAT), FEAT)
            O = cons12(M, ex, cnt)
            h = O.reshape(NPAD, FEAT)[:N]
        else:
            w2p = jnp.pad(W2, ((0, 0), (0, FEAT - IN_DIM)))
            b2p = jnp.pad(b2, (0, FEAT - IN_DIM)).reshape(1, FEAT)
            M = _mm_call(P, w2p, b2p, FEAT)
            O = cons3(M, ex, cnt, scale_flat)
            out = O.reshape(NPAD, FEAT)[:N, :IN_DIM]
    return out


# speedup vs baseline: 1.0026x; 1.0026x over previous
import math

import jax
import jax.numpy as jnp
from jax import lax
from jax.experimental import pallas as pl
from jax.experimental.pallas import tpu as pltpu
from jax.experimental.pallas import tpu_sc as plsc

N = 10000
E = 320000
FEAT = 128
IN_DIM = 7
INST = 20
F_CAT = FEAT * 3 + INST
SIGMA = 25.0

NC = 2
NS = 16
LANES = 16
NW = NC * NS

EW = E // NW
GCH = 80
NGCH = EW // GCH

NB = 320
NPAD = NW * NB
SUB = 128
WCH = 2000
PAGE = 512
PITCH = PAGE + 16
CAPB = 10240
QMUL = 6554
QSH = 21

_ROWBLK = 2000
_EBLK = 4000


def _p0_body(t_ref, x_ref, wg_ref, tw_ref, tb_ref, xw_ref, xb_ref,
             te_ref, h0_ref, sc_ref):
    t = t_ref[...]
    proj = t * wg_ref[...] * (2.0 * math.pi)
    gf = jnp.concatenate([jnp.sin(proj), jnp.cos(proj)], axis=1)
    te = jnp.dot(gf, tw_ref[...], preferred_element_type=jnp.float32) + tb_ref[...]
    te_ref[...] = te * jax.nn.sigmoid(te)
    h0_ref[...] = (jnp.dot(x_ref[...], xw_ref[...],
                           preferred_element_type=jnp.float32) + xb_ref[...])
    ln2 = 2.0 * math.log(SIGMA)
    std = jnp.sqrt((jnp.exp(t * ln2) - 1.0) / ln2)
    sc_ref[...] = jnp.broadcast_to(1.0 / (std + 1e-7), (t.shape[0], LANES))


def _p0_call(t, x, wg, tw, tb, xw, xb):
    grid = (N // _ROWBLK,)
    full = lambda shape: pl.BlockSpec(shape, lambda i: (0, 0))
    row = lambda w: pl.BlockSpec((_ROWBLK, w), lambda i: (i, 0))
    return pl.pallas_call(
        _p0_body,
        grid=grid,
        in_specs=[row(1), row(IN_DIM), full((1, FEAT // 2)),
                  full((FEAT, FEAT)), full((1, FEAT)),
                  full((IN_DIM, FEAT)), full((1, FEAT))],
        out_specs=[row(FEAT), row(FEAT), row(LANES)],
        out_shape=[jax.ShapeDtypeStruct((N, FEAT), jnp.float32),
                   jax.ShapeDtypeStruct((N, FEAT), jnp.float32),
                   jax.ShapeDtypeStruct((N, LANES), jnp.float32)],
    )(t, x, wg, tw, tb, xw, xb)


def _ab_body(h_ref, te_ref, pp_ref, il_ref,
             wdh_ref, wdt_ref, wdp_ref, wdi_ref, b1_ref,
             wbh_ref, wbt_ref, wbp_ref, wbi_ref,
             a_ref, b_ref):
    h = h_ref[...]
    te = te_ref[...]
    pp = pp_ref[...]
    il = il_ref[...]
    dot = lambda a, w: jnp.dot(a, w[...], preferred_element_type=jnp.float32)
    a_ref[...] = (dot(h, wdh_ref) + dot(te, wdt_ref) + dot(pp, wdp_ref)
                  + dot(il, wdi_ref) + b1_ref[...])
    b_ref[...] = (dot(h, wbh_ref) + dot(te, wbt_ref) + dot(pp, wbp_ref)
                  + dot(il, wbi_ref))


def _ab_call(h, te, pp, il, wdh, wdt, wdp, wdi, b1, wbh, wbt, wbp, wbi):
    grid = (N // _ROWBLK,)
    full = lambda shape: pl.BlockSpec(shape, lambda i: (0, 0))
    row = lambda w: pl.BlockSpec((_ROWBLK, w), lambda i: (i, 0))
    return pl.pallas_call(
        _ab_body,
        grid=grid,
        in_specs=[row(FEAT), row(FEAT), row(FEAT), row(INST),
                  full((FEAT, FEAT)), full((FEAT, FEAT)), full((FEAT, FEAT)),
                  full((INST, FEAT)), full((1, FEAT)),
                  full((FEAT, FEAT)), full((FEAT, FEAT)), full((FEAT, FEAT)),
                  full((INST, FEAT))],
        out_specs=[row(FEAT), row(FEAT)],
        out_shape=[jax.ShapeDtypeStruct((N, FEAT), jnp.float32),
                   jax.ShapeDtypeStruct((N, FEAT), jnp.float32)],
    )(h, te, pp, il, wdh, wdt, wdp, wdi, b1, wbh, wbt, wbp, wbi)


def _mm_body(p_ref, w2_ref, b2_ref, m_ref):
    m_ref[...] = (jnp.dot(p_ref[...], w2_ref[...],
                          preferred_element_type=jnp.float32) + b2_ref[...])


def _mm_call(p, w2, b2, doutp):
    grid = (E // _EBLK,)
    return pl.pallas_call(
        _mm_body,
        grid=grid,
        in_specs=[pl.BlockSpec((_EBLK, FEAT), lambda i: (i, 0)),
                  pl.BlockSpec((FEAT, doutp), lambda i: (0, 0)),
                  pl.BlockSpec((1, doutp), lambda i: (0, 0))],
        out_specs=pl.BlockSpec((_EBLK, doutp), lambda i: (i, 0)),
        out_shape=jax.ShapeDtypeStruct((E, doutp), jnp.float32),
    )(p, w2, b2)


def _gather_body(a_hbm, b_hbm, dst_hbm, src_hbm, p_hbm,
                 idxd_v, idxs_v, rowsa_v, rowsb_v, sem):
    wid = lax.axis_index("s") * NC + lax.axis_index("c")
    base = wid * EW

    def chunk(ch, carry):
        off = base + ch * GCH
        pltpu.sync_copy(dst_hbm.at[pl.ds(off, GCH)], idxd_v)
        pltpu.sync_copy(src_hbm.at[pl.ds(off, GCH)], idxs_v)
        da = pltpu.async_copy(a_hbm.at[idxd_v], rowsa_v, sem)
        db = pltpu.async_copy(b_hbm.at[idxs_v], rowsb_v, sem)
        da.wait()
        db.wait()

        def row(r, c2):
            for c in range(FEAT // LANES):
                s = pl.ds(c * LANES, LANES)
                rowsa_v[r, s] = jnp.maximum(rowsa_v[r, s] + rowsb_v[r, s], 0.0)
            return c2
        lax.fori_loop(0, GCH, row, 0)
        pltpu.sync_copy(rowsa_v, p_hbm.at[pl.ds(off, GCH)])
        return carry
    lax.fori_loop(0, NGCH, chunk, 0)


def _make_gather():
    mesh = plsc.VectorSubcoreMesh(core_axis_name="c", subcore_axis_name="s",
                                  num_cores=NC, num_subcores=NS)
    return pl.kernel(
        _gather_body,
        out_type=jax.ShapeDtypeStruct((E, FEAT), jnp.float32),
        mesh=mesh,
        scratch_types=[
            pltpu.VMEM((GCH,), jnp.int32),
            pltpu.VMEM((GCH,), jnp.int32),
            pltpu.VMEM((GCH, FEAT), jnp.float32),
            pltpu.VMEM((GCH, FEAT), jnp.float32),
            pltpu.SemaphoreType.DMA,
        ],
    )


def _bucket_body(dst_hbm, ex_hbm, cnt_hbm, dbuf, staging, cntbuf):
    wid = lax.axis_index("s") * NC + lax.axis_index("c")
    ebase = wid * EW
    iota = lax.iota(jnp.int32, LANES)
    zeros16 = jnp.zeros((LANES,), jnp.int32)
    for c in range(3):
        cntbuf[pl.ds(c * LANES, LANES)] = zeros16

    def chunk(ch, carry):
        pltpu.sync_copy(dst_hbm.at[pl.ds(ebase + ch * WCH, WCH)], dbuf)

        def group(g, c2):
            d16 = dbuf[pl.ds(g * LANES, LANES)]
            q16 = (d16 * QMUL) >> QSH
            dloc16 = d16 - q16 * NB
            e0 = ebase + ch * WCH + g * LANES
            pk16 = dloc16 + (e0 + iota) * 512
            for l in range(LANES):
                b = q16[l]
                pk = zeros16 + pk16[l]
                cb = cntbuf[pl.ds(b, LANES)][0]
                pos = cb & (PAGE - 1)
                staging[pl.ds(b * PITCH + pos, LANES)] = pk
                bb = (b >> 4) << 4
                cv = cntbuf[pl.ds(bb, LANES)]
                cntbuf[pl.ds(bb, LANES)] = cv + jnp.where(
                    iota == b - bb, jnp.ones((LANES,), jnp.int32), zeros16)

                @pl.when(pos == PAGE - 1)
                def _flush():
                    pltpu.sync_copy(
                        staging.at[pl.ds(b * PITCH, PAGE)],
                        ex_hbm.at[pl.ds(pl.multiple_of(
                            (wid * NW + b) * CAPB + cb - (PAGE - 1), 8),
                            PAGE)])
            return c2
        lax.fori_loop(0, WCH // LANES, group, 0)
        return carry
    lax.fori_loop(0, EW // WCH, chunk, 0)

    for b in range(NW):
        cb = cntbuf[pl.ds(b, LANES)][0]
        base = (cb >> 9) << 9
        pltpu.sync_copy(
            staging.at[pl.ds(b * PITCH, PAGE)],
            ex_hbm.at[pl.ds(pl.multiple_of(
                (wid * NW + b) * CAPB + base, 8), PAGE)])
    pltpu.sync_copy(cntbuf.at[pl.ds(0, NW)],
                    cnt_hbm.at[pl.ds(pl.multiple_of(wid * NW, 8), NW)])


def _make_bucket():
    mesh = plsc.VectorSubcoreMesh(core_axis_name="c", subcore_axis_name="s",
                                  num_cores=NC, num_subcores=NS)
    return pl.kernel(
        _bucket_body,
        out_type=(jax.ShapeDtypeStruct((NW * NW * CAPB,), jnp.int32),
                  jax.ShapeDtypeStruct((NW * NW,), jnp.int32)),
        mesh=mesh,
        scratch_types=[
            pltpu.VMEM((WCH,), jnp.int32),
            pltpu.VMEM((NW * PITCH,), jnp.int32),
            pltpu.VMEM((3 * LANES,), jnp.int32),
        ],
    )


def _consumer_body(doutp, do_relu, use_scale, *refs):
    if use_scale:
        (m_hbm, ex_hbm, cnt_hbm, sc_hbm, o_hbm,
         cntv, pkbuf, idx0, idx1, dlb0, dlb1, rows0, rows1, acc_v, scv,
         sem0, sem1) = refs
    else:
        (m_hbm, ex_hbm, cnt_hbm, o_hbm,
         cntv, pkbuf, idx0, idx1, dlb0, dlb1, rows0, rows1, acc_v,
         sem0, sem1) = refs
    wid = lax.axis_index("s") * NC + lax.axis_index("c")
    lo = wid * NB
    cg = doutp // LANES
    iota = lax.iota(jnp.int32, LANES)
    neg = jnp.full((LANES,), -jnp.inf, jnp.float32)
    slots = ((idx0, dlb0, rows0, sem0), (idx1, dlb1, rows1, sem1))

    def initr(r, c2):
        acc_v[pl.ds(r * LANES, LANES)] = neg
        return c2
    lax.fori_loop(0, (NB + 1) * cg, initr, 0)

    pltpu.sync_copy(cnt_hbm, cntv.at[pl.ds(0, NW * NW)])

    def writer(t, c2):
        ct = cntv[pl.ds(t * NW + wid, LANES)][0]
        npages = (ct + PAGE - 1) >> 9

        def loadpage(pg, c3):
            pltpu.sync_copy(
                ex_hbm.at[pl.ds(pl.multiple_of(
                    (t * NW + wid) * CAPB + pg * PAGE, 8), PAGE)],
                pkbuf.at[pl.ds(pg * PAGE, PAGE)])
            return c3
        lax.fori_loop(0, npages, loadpage, 0)

        nsubw = (ct + SUB - 1) >> 7

        def prep(sb, par):
            idxb, dlbb, rowsb, semb = slots[par]
            base = sb * SUB
            valid = ct - base
            for gg in range(SUB // LANES):
                pk = pkbuf[pl.ds(base + gg * LANES, LANES)]
                okm = (iota + gg * LANES) < valid
                idxb[pl.ds(gg * LANES, LANES)] = jnp.where(okm, pk >> 9, 0)
                dlbb[pl.ds(gg * LANES, LANES)] = jnp.where(okm, pk & 511, NB)
            pltpu.async_copy(m_hbm.at[idxb], rowsb, semb)

        @pl.when(nsubw > 0)
        def _prime():
            prep(0, 0)

        def sub(sb, c4):
            for par in range(2):
                @pl.when((sb & 1) == par)
                def _run():
                    idxb, dlbb, rowsb, semb = slots[par]

                    @pl.when(sb + 1 < nsubw)
                    def _next():
                        prep(sb + 1, 1 - par)
                    pltpu.make_async_copy(m_hbm.at[idxb], rowsb, semb).wait()

                    def k8(kk, c5):
                        dlv = dlbb[pl.ds(kk * LANES, LANES)]
                        for l in range(LANES):
                            dl = dlv[l]
                            ro = kk * LANES + l
                            for c in range(cg):
                                off = dl * doutp + c * LANES
                                cs = pl.ds(c * LANES, LANES)
                                acc_v[pl.ds(off, LANES)] = jnp.maximum(
                                    acc_v[pl.ds(off, LANES)], rowsb[ro, cs])
                        return c5
                    lax.fori_loop(0, SUB // LANES, k8, 0)
            return c4
        lax.fori_loop(0, nsubw, sub, 0)
        return c2
    lax.fori_loop(0, NW, writer, 0)

    if use_scale:
        pltpu.sync_copy(
            sc_hbm.at[pl.ds(pl.multiple_of(lo * LANES, 8), NB * LANES)], scv)

    def epi(r, c2):
        for c in range(cg):
            off = r * doutp + c * LANES
            v = acc_v[pl.ds(off, LANES)]
            ok = jnp.abs(v) < jnp.inf
            v = jnp.where(ok, v, 0.0)
            if do_relu:
                v = jnp.maximum(v, 0.0)
            if use_scale:
                v = v * scv[pl.ds(r * LANES, LANES)]
            acc_v[pl.ds(off, LANES)] = v
        return c2
    lax.fori_loop(0, NB, epi, 0)
    pltpu.sync_copy(acc_v.at[pl.ds(0, NB * doutp)],
                    o_hbm.at[pl.ds(pl.multiple_of(lo * doutp, 8),
                                   NB * doutp)])


def _make_consumer(doutp, do_relu, use_scale):
    mesh = plsc.VectorSubcoreMesh(core_axis_name="c", subcore_axis_name="s",
                                  num_cores=NC, num_subcores=NS)
    scratch = [
        pltpu.VMEM((NW * NW + LANES,), jnp.int32),
        pltpu.VMEM((CAPB,), jnp.int32),
        pltpu.VMEM((SUB,), jnp.int32),
        pltpu.VMEM((SUB,), jnp.int32),
        pltpu.VMEM((SUB,), jnp.int32),
        pltpu.VMEM((SUB,), jnp.int32),
        pltpu.VMEM((SUB, doutp), jnp.float32),
        pltpu.VMEM((SUB, doutp), jnp.float32),
        pltpu.VMEM(((NB + 1) * doutp,), jnp.float32),
    ]
    if use_scale:
        scratch.append(pltpu.VMEM((NB * LANES,), jnp.float32))
    scratch += [pltpu.SemaphoreType.DMA, pltpu.SemaphoreType.DMA]

    def body(*refs):
        _consumer_body(doutp, do_relu, use_scale, *refs)

    return pl.kernel(
        body,
        out_type=jax.ShapeDtypeStruct((NPAD * doutp,), jnp.float32),
        mesh=mesh,
        scratch_types=scratch,
    )


def kernel(x, t, proc_part_pcs, instance_label, edge_index, W_gfp, tW, tb,
           xW, xb, m1W1, m1b1, m1W2, m1b2, m2W1, m2b1, m2W2, m2b2,
           m3W1, m3b1, m3W2, m3b2):
    src = edge_index[0]
    dst = edge_index[1]
    wg = W_gfp.reshape(1, FEAT // 2)
    te, h0, scale16 = _p0_call(t, x, wg, tW, tb.reshape(1, FEAT),
                               xW, xb.reshape(1, FEAT))
    scale_pad = jnp.pad(scale16, ((0, NPAD - N), (0, 0)))

    gather = _make_gather()
    bucket = _make_bucket()
    cons12 = _make_consumer(FEAT, True, False)
    cons3 = _make_consumer(FEAT, False, True)

    ex, cnt = bucket(dst)
    scale_flat = scale_pad.reshape(-1)

    h = h0
    layers = [(m1W1, m1b1, m1W2, m1b2), (m2W1, m2b1, m2W2, m2b2),
              (m3W1, m3b1, m3W2, m3b2)]
    out = None
    for li, (W1, b1, W2, b2) in enumerate(layers):
        Wa, Wb = W1[:F_CAT], W1[F_CAT:]
        Wd = Wa - Wb
        wdh, wdt, wdp, wdi = (Wd[:FEAT], Wd[FEAT:2 * FEAT],
                              Wd[2 * FEAT:3 * FEAT], Wd[3 * FEAT:])
        wbh, wbt, wbp, wbi = (Wb[:FEAT], Wb[FEAT:2 * FEAT],
                              Wb[2 * FEAT:3 * FEAT], Wb[3 * FEAT:])
        A, B = _ab_call(h, te, proc_part_pcs, instance_label,
                        wdh, wdt, wdp, wdi, b1.reshape(1, FEAT),
                        wbh, wbt, wbp, wbi)
        P = gather(A, B, dst, src)
        if li < 2:
            M = _mm_call(P, W2, b2.reshape(1, FEAT), FEAT)
            O = cons12(M, ex, cnt)
            h = O.reshape(NPAD, FEAT)[:N]
        else:
            w2p = jnp.pad(W2, ((0, 0), (0, FEAT - IN_DIM)))
            b2p = jnp.pad(b2, (0, FEAT - IN_DIM)).reshape(1, FEAT)
            M = _mm_call(P, w2p, b2p, FEAT)
            O = cons3(M, ex, cnt, scale_flat)
            out = O.reshape(NPAD, FEAT)[:N, :IN_DIM]
    return out
